# baseline (speedup 1.0000x reference)
import jax
import jax.numpy as jnp
from jax import lax
from jax.experimental import pallas as pl
from jax.experimental.pallas import tpu as pltpu
from jax.experimental.pallas import tpu_sc as plsc

VOCAB = 1000000
D = 32
B = 16384
H = 200

NC, NS = 2, 16
NW = NC * NS

BW = 256
NBQ = B // BW
NHB = H // 8
NSU = NHB * NBQ
SU_PER_W = NSU // NW
STEPS = SU_PER_W * 8
OUT_ROWS = H * 4 * 128 * 8


def _body(idx_hbm, tab_hbm, out_hbm,
          it, ridx0, ridx1, ridx2, gb0, gb1, gb2, tb0, tb1, tb2,
          s_g0, s_g1, s_g2, s_o0, s_o1, s_o2):
  cid = lax.axis_index("c")
  sid = lax.axis_index("s")
  wid = sid * NC + cid
  iota = lax.iota(jnp.int32, 16)
  ridx = [ridx0, ridx1, ridx2]
  gb = [gb0, gb1, gb2]
  tb = [tb0, tb1, tb2]
  s_g = [s_g0, s_g1, s_g2]
  s_o = [s_o0, s_o1, s_o2]

  def unit_coords(g2):
    dh = lax.rem(g2, 8)
    su = wid * SU_PER_W + lax.div(g2, 8)
    hblk = lax.div(su, NBQ)
    bq = lax.rem(su, NBQ)
    return dh, hblk, bq

  def build_ridx(g2, b):
    dh, hblk, bq = unit_coords(g2)

    @pl.when(dh == 0)
    def _():
      pltpu.sync_copy(
          idx_hbm.at[pl.ds(8 * hblk, 8), pl.ds(BW * bq, BW)], it)

    for k in range(BW // 16):
      g = plsc.load_gather(
          it, [jnp.broadcast_to(dh, (16,)), iota + 16 * k])
      ridx[b][pl.ds(16 * k, 16)] = g

  def consume(p, pb):
    dh, hblk, bq = unit_coords(p)
    h = 8 * hblk + dh
    pltpu.make_async_copy(tab_hbm.at[ridx[pb]], gb[pb], s_g[pb]).wait()

    @pl.when(p >= 3)
    def _():
      for _ in range(4):
        pltpu.make_async_copy(
            tb[pb].at[pl.ds(0, 2048)],
            out_hbm.at[pl.ds(0, 2048)], s_o[pb]).wait()

    for d in range(D):
      dhi, dlo = d // 8, d % 8
      gs = [
          plsc.load_gather(
              gb[pb], [iota + 16 * t, jnp.full((16,), d, jnp.int32)])
          for t in range(BW // 16)
      ]
      for t in range(BW // 16):
        s, k2 = t // 8, t % 8
        tb[pb][pl.ds(dhi * 2048 + (s * 8 + dlo) * 128 + 16 * k2, 16)] = gs[t]

    for dhi in range(4):
      off = (h * 4096 + dhi * 1024 + 16 * bq) * 128
      pltpu.async_copy(
          tb[pb].at[pl.ds(dhi * 2048, 2048)],
          out_hbm.at[pl.ds(off, 2048)], s_o[pb])

  def p2_body(i, _):
    for b in range(3):
      g2 = 3 * i + b

      @pl.when(g2 >= 3)
      def _():
        consume(g2 - 3, b)

      build_ridx(g2, b)
      pltpu.async_copy(tab_hbm.at[ridx[b]], gb[b], s_g[b])
    return ()

  lax.fori_loop(0, (STEPS - 1) // 3, p2_body, ())

  consume(STEPS - 4, 0)
  build_ridx(STEPS - 1, 0)
  pltpu.async_copy(tab_hbm.at[ridx[0]], gb[0], s_g[0])
  consume(STEPS - 3, 1)
  consume(STEPS - 2, 2)
  consume(STEPS - 1, 0)
  for b in range(3):
    for _ in range(4):
      pltpu.make_async_copy(
          tb[b].at[pl.ds(0, 2048)],
          out_hbm.at[pl.ds(0, 2048)], s_o[b]).wait()


@jax.jit
def _lookup(idx_t, embeddings):
  mesh = plsc.VectorSubcoreMesh(core_axis_name="c", subcore_axis_name="s")
  f = pl.kernel(
      _body,
      out_type=jax.ShapeDtypeStruct((OUT_ROWS * 128,), jnp.float32),
      mesh=mesh,
      scratch_types=[
          pltpu.VMEM((8, BW), jnp.int32),
          pltpu.VMEM((BW,), jnp.int32),
          pltpu.VMEM((BW,), jnp.int32),
          pltpu.VMEM((BW,), jnp.int32),
          pltpu.VMEM((BW, D), jnp.float32),
          pltpu.VMEM((BW, D), jnp.float32),
          pltpu.VMEM((BW, D), jnp.float32),
          pltpu.VMEM((4 * 16 * 128,), jnp.float32),
          pltpu.VMEM((4 * 16 * 128,), jnp.float32),
          pltpu.VMEM((4 * 16 * 128,), jnp.float32),
          pltpu.SemaphoreType.DMA,
          pltpu.SemaphoreType.DMA,
          pltpu.SemaphoreType.DMA,
          pltpu.SemaphoreType.DMA,
          pltpu.SemaphoreType.DMA,
          pltpu.SemaphoreType.DMA,
      ],
      compiler_params=pltpu.CompilerParams(
          use_tc_tiling_on_sc=False, needs_layout_passes=False),
  )
  return f(idx_t, embeddings)


def kernel(inputs, embeddings):
  idx_t = jnp.transpose(inputs).astype(jnp.int32)
  out5 = _lookup(idx_t, embeddings)
  o = jnp.reshape(out5, (H, 4, 128, 8, 128))
  o = jnp.transpose(o, (2, 4, 0, 1, 3))
  return jnp.reshape(o, (B, H, D))


# speedup vs baseline: 1.4846x; 1.4846x over previous
import jax
import jax.numpy as jnp
from jax import lax
from jax.experimental import pallas as pl
from jax.experimental.pallas import tpu as pltpu
from jax.experimental.pallas import tpu_sc as plsc

VOCAB = 1000000
D = 32
B = 16384
H = 200

NC, NS = 2, 16
NW = NC * NS

BW = 256
NBQ = B // BW
NHB = H // 8
NSU = NHB * NBQ
SU_PER_W = NSU // NW
STEPS = SU_PER_W * 8
OUT_ROWS = H * 4 * 128 * 8


def _body(idx_hbm, tab_hbm, out_hbm,
          it, ridx0, ridx1, gb0, gb1, gbp, tb0, tb1,
          s_g0, s_g1, s_o0, s_o1):
  cid = lax.axis_index("c")
  sid = lax.axis_index("s")
  wid = sid * NC + cid
  iota = lax.iota(jnp.int32, 16)
  i33 = iota * 33
  ridx = [ridx0, ridx1]
  gb = [gb0, gb1]
  tb = [tb0, tb1]
  s_g = [s_g0, s_g1]
  s_o = [s_o0, s_o1]

  def unit_coords(g2):
    dh = lax.rem(g2, 8)
    su = wid * SU_PER_W + lax.div(g2, 8)
    hblk = lax.div(su, NBQ)
    bq = lax.rem(su, NBQ)
    return dh, hblk, bq

  def build_ridx(g2, b):
    dh, hblk, bq = unit_coords(g2)

    @pl.when(dh == 0)
    def _():
      pltpu.sync_copy(
          idx_hbm.at[pl.ds(8 * hblk, 8), pl.ds(BW * bq, BW)], it)

    for k in range(BW // 16):
      g = plsc.load_gather(
          it, [jnp.broadcast_to(dh, (16,)), iota + 16 * k])
      ridx[b][pl.ds(16 * k, 16)] = g

  def consume(p, pb):
    dh, hblk, bq = unit_coords(p)
    h = 8 * hblk + dh
    pltpu.make_async_copy(tab_hbm.at[ridx[pb]], gb[pb], s_g[pb]).wait()

    @pl.when(p >= 2)
    def _():
      for _ in range(4):
        pltpu.make_async_copy(
            tb[pb].at[pl.ds(0, 2048)],
            out_hbm.at[pl.ds(0, 2048)], s_o[pb]).wait()

    for j in range(BW):
      for h2 in range(2):
        gbp[pl.ds(33 * j + 16 * h2, 16)] = gb[pb][j, pl.ds(16 * h2, 16)]

    for d in range(D):
      dhi, dlo = d // 8, d % 8
      gs = [
          plsc.load_gather(gbp, [i33 + (33 * 16 * t + d)])
          for t in range(BW // 16)
      ]
      for t in range(BW // 16):
        s, k2 = t // 8, t % 8
        tb[pb][pl.ds(dhi * 2048 + (s * 8 + dlo) * 128 + 16 * k2, 16)] = gs[t]

    for dhi in range(4):
      off = (h * 4096 + dhi * 1024 + 16 * bq) * 128
      pltpu.async_copy(
          tb[pb].at[pl.ds(dhi * 2048, 2048)],
          out_hbm.at[pl.ds(off, 2048)], s_o[pb])

  def p2_body(i, _):
    for b in range(2):
      g2 = 2 * i + b

      @pl.when(g2 >= 2)
      def _():
        consume(g2 - 2, b)

      build_ridx(g2, b)
      pltpu.async_copy(tab_hbm.at[ridx[b]], gb[b], s_g[b])
    return ()

  lax.fori_loop(0, STEPS // 2, p2_body, ())

  for b in range(2):
    consume(STEPS - 2 + b, b)
  for b in range(2):
    for _ in range(4):
      pltpu.make_async_copy(
          tb[b].at[pl.ds(0, 2048)],
          out_hbm.at[pl.ds(0, 2048)], s_o[b]).wait()


@jax.jit
def _lookup(idx_t, embeddings):
  mesh = plsc.VectorSubcoreMesh(core_axis_name="c", subcore_axis_name="s")
  f = pl.kernel(
      _body,
      out_type=jax.ShapeDtypeStruct((OUT_ROWS * 128,), jnp.float32),
      mesh=mesh,
      scratch_types=[
          pltpu.VMEM((8, BW), jnp.int32),
          pltpu.VMEM((BW,), jnp.int32),
          pltpu.VMEM((BW,), jnp.int32),
          pltpu.VMEM((BW, D), jnp.float32),
          pltpu.VMEM((BW, D), jnp.float32),
          pltpu.VMEM((BW * 33,), jnp.float32),
          pltpu.VMEM((4 * 16 * 128,), jnp.float32),
          pltpu.VMEM((4 * 16 * 128,), jnp.float32),
          pltpu.SemaphoreType.DMA,
          pltpu.SemaphoreType.DMA,
          pltpu.SemaphoreType.DMA,
          pltpu.SemaphoreType.DMA,
      ],
      compiler_params=pltpu.CompilerParams(
          use_tc_tiling_on_sc=False, needs_layout_passes=False),
  )
  return f(idx_t, embeddings)


def kernel(inputs, embeddings):
  idx_t = jnp.transpose(inputs).astype(jnp.int32)
  out5 = _lookup(idx_t, embeddings)
  o = jnp.reshape(out5, (H, 4, 128, 8, 128))
  o = jnp.transpose(o, (2, 4, 0, 1, 3))
  return jnp.reshape(o, (B, H, D))
